# baseline (speedup 1.0000x reference)
import jax
import jax.numpy as jnp
from jax.experimental import pallas as pl

N = 100000
NODE_TYPE_SIZE = 9


def _add_kernel(a_ref, b_ref, o_ref):
    o_ref[...] = a_ref[...] + b_ref[...]


def kernel(x, node_type, edge_index, mesh_pos, xtime, We1, be1, We2, be2, Wn1, bn1, Wo, bo):
    nt_oh = jax.nn.one_hot(node_type[:, 0], NODE_TYPE_SIZE, dtype=jnp.float32)
    node_features = jnp.concatenate([x, nt_oh], axis=-1)
    senders = edge_index[:, 0]
    receivers = edge_index[:, 1]
    rel = mesh_pos[senders] - mesh_pos[receivers]
    edge_features = jnp.concatenate(
        [rel, jnp.linalg.norm(rel, axis=-1, keepdims=True)], axis=-1)
    ef = jnp.concatenate(
        [edge_features, node_features[senders], node_features[receivers]], axis=-1)
    m = jax.nn.relu(ef @ We1 + be1)
    m = jax.nn.relu(m @ We2 + be2)
    agg = jax.ops.segment_sum(m, receivers, num_segments=N)
    h = jax.nn.relu(jnp.concatenate([node_features, agg], axis=-1) @ Wn1 + bn1)
    per_node_out = h @ Wo + bo
    cur_x = xtime[..., -1]
    blk = 10000
    return pl.pallas_call(
        _add_kernel,
        out_shape=jax.ShapeDtypeStruct(cur_x.shape, cur_x.dtype),
        grid=(N // blk,),
        in_specs=[pl.BlockSpec((blk, 2), lambda i: (i, 0)),
                  pl.BlockSpec((blk, 2), lambda i: (i, 0))],
        out_specs=pl.BlockSpec((blk, 2), lambda i: (i, 0)),
    )(cur_x, per_node_out)


# speedup vs baseline: 21.4310x; 21.4310x over previous
import functools

import numpy as np
import jax
import jax.numpy as jnp
from jax import lax
from jax.experimental import pallas as pl
from jax.experimental.pallas import tpu as pltpu
import jax.experimental.pallas.tpu_sc as plsc

N = 100000
E = 6400000
NODE_TYPE_SIZE = 9
HID = 16

NC = 2
NS = 16
NW = NC * NS

G = 128
GPC = 16
NCHUNK = 98
GROUPS_PT = GPC * NCHUNK
EPT = GROUPS_PT * G
E_PAD = EPT * NW
N_PAD = 102400
RB = 2048
ROWS_PT = N_PAD // NS
WCHUNK = 200


def _tables_body(x_ref, nt_ref, pos_ref, w1a_ref, w1b_ref, w1p_ref, be1_ref,
                 ta_ref, tb_ref):
    xv = x_ref[...]
    nt = nt_ref[...]
    oh = (nt == lax.broadcasted_iota(jnp.int32, (RB, NODE_TYPE_SIZE), 1))
    nf = jnp.concatenate([xv, oh.astype(jnp.float32)], axis=1)
    pos = pos_ref[...]
    q = jnp.dot(pos, w1p_ref[...], preferred_element_type=jnp.float32)
    pa = jnp.dot(nf, w1a_ref[...], preferred_element_type=jnp.float32) + q + be1_ref[...]
    pb = jnp.dot(nf, w1b_ref[...], preferred_element_type=jnp.float32) - q
    post = jnp.tile(pos, (1, 8))
    ta_ref[...] = jnp.concatenate([pa, post], axis=1)
    tb_ref[...] = jnp.concatenate([pb, post], axis=1)


def _build_tables(x_p, nt_p, pos_p, w1a, w1b, w1p, be1r):
    grid = (N_PAD // RB,)
    return pl.pallas_call(
        _tables_body,
        out_shape=(jax.ShapeDtypeStruct((N_PAD, 32), jnp.float32),
                   jax.ShapeDtypeStruct((N_PAD, 32), jnp.float32)),
        grid=grid,
        in_specs=[pl.BlockSpec((RB, 2), lambda i: (i, 0)),
                  pl.BlockSpec((RB, 1), lambda i: (i, 0)),
                  pl.BlockSpec((RB, 2), lambda i: (i, 0)),
                  pl.BlockSpec((11, HID), lambda i: (0, 0)),
                  pl.BlockSpec((11, HID), lambda i: (0, 0)),
                  pl.BlockSpec((2, HID), lambda i: (0, 0)),
                  pl.BlockSpec((1, HID), lambda i: (0, 0))],
        out_specs=(pl.BlockSpec((RB, 32), lambda i: (i, 0)),
                   pl.BlockSpec((RB, 32), lambda i: (i, 0))),
    )(x_p, nt_p, pos_p, w1a, w1b, w1p, be1r)


_MAGIC = np.int32(0x5F3759DF)

_GDN = lax.GatherDimensionNumbers(
    offset_dims=(), collapsed_slice_dims=(0,), start_index_map=(0,))


def _lane_bcast(v, idx):
    return lax.gather(v, idx[:, None], _GDN, (1,),
                      mode=lax.GatherScatterMode.PROMISE_IN_BOUNDS)


def _edge_body(s2d, r2d, ta, tb, we2, cons, out,
               sidx, ridx, buf_a, buf_b, mbuf, we2v, cv, zb, agg_sh,
               sem_a, sem_b):
    cid = lax.axis_index("c")
    sid = lax.axis_index("s")
    wid = sid * NC + cid

    pltpu.sync_copy(we2, we2v)
    pltpu.sync_copy(cons, cv)

    iot = lax.iota(jnp.int32, 16)
    swap = iot ^ 1
    zeros16 = (iot ^ iot).astype(jnp.float32)

    @pl.loop(0, WCHUNK)
    def _z(i):
        zb[i, :] = zeros16

    @pl.loop(0, ROWS_PT // WCHUNK)
    def _zs(i):
        pltpu.sync_copy(zb, agg_sh.at[pl.ds(sid * ROWS_PT + i * WCHUNK, WCHUNK)])

    plsc.subcore_barrier()

    w2 = cv[0, :]
    be2v = cv[1, :]
    we2_rows = [we2v[k, :] for k in range(HID)]
    zero_i = iot & 0
    kidx = [zero_i + k for k in range(HID)]

    @pl.loop(0, NCHUNK)
    def _chunk(c):
        gbase = wid * GROUPS_PT + c * GPC
        pltpu.sync_copy(s2d.at[pl.ds(gbase, GPC)], sidx)
        pltpu.sync_copy(r2d.at[pl.ds(gbase, GPC)], ridx)

        @pl.loop(0, GPC)
        def _grp(g):
            cp_a = pltpu.async_copy(ta.at[sidx.at[g]], buf_a, sem_a)
            cp_b = pltpu.async_copy(tb.at[ridx.at[g]], buf_b, sem_b)
            cp_a.wait()
            cp_b.wait()

            @plsc.parallel_loop(0, G, unroll=2)
            def _edge(e):
                a0 = buf_a[e, pl.ds(0, 16)]
                a1 = buf_a[e, pl.ds(16, 16)]
                b0 = buf_b[e, pl.ds(0, 16)]
                b1 = buf_b[e, pl.ds(16, 16)]
                d = a1 - b1
                dd = d * d
                nsq = dd + _lane_bcast(dd, swap)
                y = lax.bitcast_convert_type(
                    _MAGIC - (lax.bitcast_convert_type(nsq, jnp.int32) >> 1),
                    jnp.float32)
                h = 0.5 * nsq
                y = y * (1.5 - h * y * y)
                y = y * (1.5 - h * y * y)
                y = y * (1.5 - h * y * y)
                norm = nsq * y
                m1 = jnp.maximum(a0 + b0 + norm * w2, 0.0)
                acc = be2v
                for k in range(HID):
                    acc = acc + _lane_bcast(m1, kidx[k]) * we2_rows[k]
                mbuf[e, :] = jnp.maximum(acc, 0.0)

            pltpu.sync_copy(mbuf, agg_sh.at[ridx.at[g]], add=True)

    plsc.subcore_barrier()

    @pl.loop(0, ROWS_PT // WCHUNK)
    def _w(i):
        row = sid * ROWS_PT + i * WCHUNK
        pltpu.sync_copy(agg_sh.at[pl.ds(row, WCHUNK)], zb)
        pltpu.sync_copy(zb, out.at[cid, pl.ds(row, WCHUNK)])


_edge_kernel = functools.partial(
    pl.kernel,
    out_type=jax.ShapeDtypeStruct((NC, N_PAD, HID), jnp.float32),
    mesh=plsc.VectorSubcoreMesh(core_axis_name="c", subcore_axis_name="s"),
    compiler_params=pltpu.CompilerParams(use_tc_tiling_on_sc=False),
    scratch_types=[
        pltpu.VMEM((GPC, G), jnp.int32),
        pltpu.VMEM((GPC, G), jnp.int32),
        pltpu.VMEM((G, 32), jnp.float32),
        pltpu.VMEM((G, 32), jnp.float32),
        pltpu.VMEM((G, HID), jnp.float32),
        pltpu.VMEM((HID, HID), jnp.float32),
        pltpu.VMEM((2, HID), jnp.float32),
        pltpu.VMEM((WCHUNK, HID), jnp.float32),
        pltpu.VMEM_SHARED((N_PAD, HID), jnp.float32),
        pltpu.SemaphoreType.DMA,
        pltpu.SemaphoreType.DMA,
    ])(_edge_body)


def _node_body(x_ref, nt_ref, agg_a_ref, agg_b_ref, cx_ref,
               wn1a_ref, wn1b_ref, bn1_ref, wo_ref, bo_ref, out_ref):
    xv = x_ref[...]
    nt = nt_ref[...]
    oh = (nt == lax.broadcasted_iota(jnp.int32, (RB, NODE_TYPE_SIZE), 1))
    nf = jnp.concatenate([xv, oh.astype(jnp.float32)], axis=1)
    agg = agg_a_ref[...] + agg_b_ref[...]
    h = jnp.dot(nf, wn1a_ref[...], preferred_element_type=jnp.float32)
    h = h + jnp.dot(agg, wn1b_ref[...], preferred_element_type=jnp.float32)
    h = jnp.maximum(h + bn1_ref[...], 0.0)
    out_ref[...] = (jnp.dot(h, wo_ref[...], preferred_element_type=jnp.float32)
                    + bo_ref[...] + cx_ref[...])


def _node_mlp(x_p, nt_p, agg_a, agg_b, cx_p, wn1a, wn1b, bn1r, wo, bor):
    grid = (N_PAD // RB,)
    return pl.pallas_call(
        _node_body,
        out_shape=jax.ShapeDtypeStruct((N_PAD, 2), jnp.float32),
        grid=grid,
        in_specs=[pl.BlockSpec((RB, 2), lambda i: (i, 0)),
                  pl.BlockSpec((RB, 1), lambda i: (i, 0)),
                  pl.BlockSpec((RB, HID), lambda i: (i, 0)),
                  pl.BlockSpec((RB, HID), lambda i: (i, 0)),
                  pl.BlockSpec((RB, 2), lambda i: (i, 0)),
                  pl.BlockSpec((11, HID), lambda i: (0, 0)),
                  pl.BlockSpec((HID, HID), lambda i: (0, 0)),
                  pl.BlockSpec((1, HID), lambda i: (0, 0)),
                  pl.BlockSpec((HID, 2), lambda i: (0, 0)),
                  pl.BlockSpec((1, 2), lambda i: (0, 0))],
        out_specs=pl.BlockSpec((RB, 2), lambda i: (i, 0)),
    )(x_p, nt_p, agg_a, agg_b, cx_p, wn1a, wn1b, bn1r, wo, bor)


def kernel(x, node_type, edge_index, mesh_pos, xtime,
           We1, be1, We2, be2, Wn1, bn1, Wo, bo):
    f32 = jnp.float32
    npad = N_PAD - N
    x_p = jnp.concatenate([x, jnp.zeros((npad, 2), f32)])
    nt_p = jnp.concatenate([node_type, jnp.zeros((npad, 1), jnp.int32)])
    pos_p = jnp.concatenate([mesh_pos, jnp.zeros((npad, 2), f32)])
    cx_p = jnp.concatenate([xtime[..., -1], jnp.zeros((npad, 2), f32)])

    epad = E_PAD - E
    s2d = jnp.concatenate([edge_index[:, 0],
                           jnp.zeros((epad,), jnp.int32)]).reshape(E_PAD // G, G)
    r2d = jnp.concatenate([edge_index[:, 1],
                           jnp.full((epad,), N, jnp.int32)]).reshape(E_PAD // G, G)

    w1p = We1[0:2]
    w1a = We1[3:14]
    w1b = We1[14:25]
    cons = jnp.stack([We1[2], be2])

    ta, tb = _build_tables(x_p, nt_p, pos_p, w1a, w1b, w1p, be1.reshape(1, HID))
    agg = _edge_kernel(s2d, r2d, ta, tb, We2, cons)
    out = _node_mlp(x_p, nt_p, agg[0], agg[1], cx_p,
                    Wn1[:11], Wn1[11:], bn1.reshape(1, HID), Wo, bo.reshape(1, 2))
    return out[:N]


# speedup vs baseline: 30.2513x; 1.4116x over previous
import functools

import numpy as np
import jax
import jax.numpy as jnp
from jax import lax
from jax.experimental import pallas as pl
from jax.experimental.pallas import tpu as pltpu
import jax.experimental.pallas.tpu_sc as plsc

N = 100000
E = 6400000
NODE_TYPE_SIZE = 9
HID = 16

NC = 2
NS = 16
NW = NC * NS

ROW = 24
G = 128
GPC = 16
NCHUNK = 98
GROUPS_PT = GPC * NCHUNK
EPT = GROUPS_PT * G
E_PAD = EPT * NW
N_PAD = 102400
AGG_ROWS = 100352
RPT = AGG_ROWS // NS
RB = 2048
NKP = 9


def _tables_body(x_ref, nt_ref, pos_ref, w1a_ref, w1b_ref, w1p_ref, be1_ref,
                 ta_ref, tb_ref):
    xv = x_ref[...]
    nt = nt_ref[...]
    oh = (nt == lax.broadcasted_iota(jnp.int32, (RB, NODE_TYPE_SIZE), 1))
    nf = jnp.concatenate([xv, oh.astype(jnp.float32)], axis=1)
    pos = pos_ref[...]
    q = jnp.dot(pos, w1p_ref[...], preferred_element_type=jnp.float32)
    pa = jnp.dot(nf, w1a_ref[...], preferred_element_type=jnp.float32) + q + be1_ref[...]
    pb = jnp.dot(nf, w1b_ref[...], preferred_element_type=jnp.float32) - q
    post = jnp.tile(pos, (1, 4))
    ta_ref[...] = jnp.concatenate([pa, post], axis=1)
    tb_ref[...] = jnp.concatenate([pb, post], axis=1)


def _build_tables(x_p, nt_p, pos_p, w1a, w1b, w1p, be1r):
    grid = (N_PAD // RB,)
    return pl.pallas_call(
        _tables_body,
        out_shape=(jax.ShapeDtypeStruct((N_PAD, ROW), jnp.float32),
                   jax.ShapeDtypeStruct((N_PAD, ROW), jnp.float32)),
        grid=grid,
        in_specs=[pl.BlockSpec((RB, 2), lambda i: (i, 0)),
                  pl.BlockSpec((RB, 1), lambda i: (i, 0)),
                  pl.BlockSpec((RB, 2), lambda i: (i, 0)),
                  pl.BlockSpec((11, HID), lambda i: (0, 0)),
                  pl.BlockSpec((11, HID), lambda i: (0, 0)),
                  pl.BlockSpec((2, HID), lambda i: (0, 0)),
                  pl.BlockSpec((1, HID), lambda i: (0, 0))],
        out_specs=(pl.BlockSpec((RB, ROW), lambda i: (i, 0)),
                   pl.BlockSpec((RB, ROW), lambda i: (i, 0))),
    )(x_p, nt_p, pos_p, w1a, w1b, w1p, be1r)


_MAGIC = np.int32(0x5F3759DF)

_GDN = lax.GatherDimensionNumbers(
    offset_dims=(), collapsed_slice_dims=(0,), start_index_map=(0,))


def _lane_bcast(v, idx):
    return lax.gather(v, idx[:, None], _GDN, (1,),
                      mode=lax.GatherScatterMode.PROMISE_IN_BOUNDS)


def _edge_body(s2d, r2d, ta, tb, we2, cons, out,
               sidx, ridx, buf_a, buf_b, mb0, mb1, we2v, cv, agg_sh,
               sa0, sa1, sb0, sb1, sm0, sm1):
    cid = lax.axis_index("c")
    sid = lax.axis_index("s")
    wid = sid * NC + cid
    sem_a = [sa0, sa1]
    sem_b = [sb0, sb1]
    sem_m = [sm0, sm1]
    mbufs = [mb0, mb1]

    pltpu.sync_copy(we2, we2v)
    pltpu.sync_copy(cons, cv)

    iot = lax.iota(jnp.int32, 16)
    swap = iot ^ 1
    zeros16 = (iot ^ iot).astype(jnp.float32)
    zero_i = iot & 0
    eight = zero_i + 8

    @pl.loop(0, G)
    def _z(i):
        mb0[i, :] = zeros16

    @pl.loop(0, RPT // G)
    def _zs(i):
        pltpu.sync_copy(mb0, agg_sh.at[pl.ds(sid * RPT + i * G, G)])

    plsc.subcore_barrier()

    w2 = cv[0, :]
    be2v = cv[1, :]
    we2_rows = [we2v[k, :] for k in range(HID)]

    def _issue(g, b):
        pltpu.async_copy(ta.at[sidx.at[g]], buf_a.at[b], sem_a[b])
        pltpu.async_copy(tb.at[ridx.at[g]], buf_b.at[b], sem_b[b])

    def _wait(b):
        pltpu.make_async_copy(ta.at[sidx.at[0]], buf_a.at[b], sem_a[b]).wait()
        pltpu.make_async_copy(tb.at[ridx.at[0]], buf_b.at[b], sem_b[b]).wait()

    def _compute(g, b, first):
        mb = mbufs[b]

        @pl.when(jnp.logical_not(first))
        def _():
            pltpu.make_async_copy(mb, agg_sh.at[ridx.at[0]], sem_m[b]).wait()

        @plsc.parallel_loop(0, G, unroll=2)
        def _edge(e):
            a0 = buf_a[b, e, pl.ds(0, 16)]
            a1 = buf_a[b, e, pl.ds(8, 16)]
            b0 = buf_b[b, e, pl.ds(0, 16)]
            b1 = buf_b[b, e, pl.ds(8, 16)]
            d = a1 - b1
            dd = d * d
            s = dd + _lane_bcast(dd, swap)
            nsq = _lane_bcast(s, eight)
            y = lax.bitcast_convert_type(
                _MAGIC - (lax.bitcast_convert_type(nsq, jnp.int32) >> 1),
                jnp.float32)
            h = 0.5 * nsq
            y = y * (1.5 - h * y * y)
            y = y * (1.5 - h * y * y)
            y = y * (1.5 - h * y * y)
            norm = nsq * y
            m1 = jnp.maximum(a0 + b0 + norm * w2, 0.0)
            acc = be2v
            for k in range(HID):
                acc = acc + _lane_bcast(m1, zero_i + k) * we2_rows[k]
            mb[e, :] = jnp.maximum(acc, 0.0)

        pltpu.async_copy(mb, agg_sh.at[ridx.at[g]], sem_m[b], add=True)

    @pl.loop(0, NCHUNK)
    def _chunk(c):
        gbase = wid * GROUPS_PT + c * GPC
        pltpu.sync_copy(s2d.at[pl.ds(gbase, GPC)], sidx)
        pltpu.sync_copy(r2d.at[pl.ds(gbase, GPC)], ridx)
        _issue(0, 0)

        @pl.loop(0, GPC // 2)
        def _pair(p):
            g0 = 2 * p
            _issue(g0 + 1, 1)
            _wait(0)
            _compute(g0, 0, (c + p) == 0)

            @pl.when(p < GPC // 2 - 1)
            def _():
                _issue(g0 + 2, 0)

            _wait(1)
            _compute(g0 + 1, 1, (c + p) == 0)

    for b in (0, 1):
        pltpu.make_async_copy(mbufs[b], agg_sh.at[ridx.at[0]], sem_m[b]).wait()

    plsc.subcore_barrier()

    @pl.loop(0, RPT // G)
    def _w(i):
        row = sid * RPT + i * G
        pltpu.sync_copy(agg_sh.at[pl.ds(row, G)], mb0)
        pltpu.sync_copy(mb0, out.at[cid, pl.ds(row, G)])


_edge_kernel = functools.partial(
    pl.kernel,
    out_type=jax.ShapeDtypeStruct((NC, AGG_ROWS, HID), jnp.float32),
    mesh=plsc.VectorSubcoreMesh(core_axis_name="c", subcore_axis_name="s"),
    compiler_params=pltpu.CompilerParams(use_tc_tiling_on_sc=False),
    scratch_types=[
        pltpu.VMEM((GPC, G), jnp.int32),
        pltpu.VMEM((GPC, G), jnp.int32),
        pltpu.VMEM((2, G, ROW), jnp.float32),
        pltpu.VMEM((2, G, ROW), jnp.float32),
        pltpu.VMEM((G, HID), jnp.float32),
        pltpu.VMEM((G, HID), jnp.float32),
        pltpu.VMEM((HID, HID), jnp.float32),
        pltpu.VMEM((2, HID), jnp.float32),
        pltpu.VMEM_SHARED((AGG_ROWS, HID), jnp.float32),
        pltpu.SemaphoreType.DMA,
        pltpu.SemaphoreType.DMA,
        pltpu.SemaphoreType.DMA,
        pltpu.SemaphoreType.DMA,
        pltpu.SemaphoreType.DMA,
        pltpu.SemaphoreType.DMA,
    ])(_edge_body)


def _node_body(x_ref, nt_ref, agg_a_ref, agg_b_ref, cx_ref,
               wn1a_ref, wn1b_ref, bn1_ref, wo_ref, bo_ref, out_ref):
    xv = x_ref[...]
    nt = nt_ref[...]
    oh = (nt == lax.broadcasted_iota(jnp.int32, (RB, NODE_TYPE_SIZE), 1))
    nf = jnp.concatenate([xv, oh.astype(jnp.float32)], axis=1)
    agg = agg_a_ref[...] + agg_b_ref[...]
    h = jnp.dot(nf, wn1a_ref[...], preferred_element_type=jnp.float32)
    h = h + jnp.dot(agg, wn1b_ref[...], preferred_element_type=jnp.float32)
    h = jnp.maximum(h + bn1_ref[...], 0.0)
    out_ref[...] = (jnp.dot(h, wo_ref[...], preferred_element_type=jnp.float32)
                    + bo_ref[...] + cx_ref[...])


def _node_mlp(x_p, nt_p, agg_a, agg_b, cx_p, wn1a, wn1b, bn1r, wo, bor):
    grid = (AGG_ROWS // RB,)
    return pl.pallas_call(
        _node_body,
        out_shape=jax.ShapeDtypeStruct((AGG_ROWS, 2), jnp.float32),
        grid=grid,
        in_specs=[pl.BlockSpec((RB, 2), lambda i: (i, 0)),
                  pl.BlockSpec((RB, 1), lambda i: (i, 0)),
                  pl.BlockSpec((RB, HID), lambda i: (i, 0)),
                  pl.BlockSpec((RB, HID), lambda i: (i, 0)),
                  pl.BlockSpec((RB, 2), lambda i: (i, 0)),
                  pl.BlockSpec((11, HID), lambda i: (0, 0)),
                  pl.BlockSpec((HID, HID), lambda i: (0, 0)),
                  pl.BlockSpec((1, HID), lambda i: (0, 0)),
                  pl.BlockSpec((HID, 2), lambda i: (0, 0)),
                  pl.BlockSpec((1, 2), lambda i: (0, 0))],
        out_specs=pl.BlockSpec((RB, 2), lambda i: (i, 0)),
    )(x_p, nt_p, agg_a, agg_b, cx_p, wn1a, wn1b, bn1r, wo, bor)


def kernel(x, node_type, edge_index, mesh_pos, xtime,
           We1, be1, We2, be2, Wn1, bn1, Wo, bo):
    f32 = jnp.float32
    npad = N_PAD - N
    x_p = jnp.concatenate([x, jnp.zeros((npad, 2), f32)])
    nt_p = jnp.concatenate([node_type, jnp.zeros((npad, 1), jnp.int32)])
    pos_p = jnp.concatenate([mesh_pos, jnp.zeros((npad, 2), f32)])
    cx_p = jnp.concatenate([xtime[..., -1], jnp.zeros((npad, 2), f32)])

    epad = E_PAD - E
    s2d = jnp.concatenate([edge_index[:, 0],
                           jnp.zeros((epad,), jnp.int32)]).reshape(E_PAD // G, G)
    r2d = jnp.concatenate([edge_index[:, 1],
                           jnp.full((epad,), N, jnp.int32)]).reshape(E_PAD // G, G)

    w1p = We1[0:2]
    w1a = We1[3:14]
    w1b = We1[14:25]
    cons = jnp.stack([We1[2], be2])

    ta, tb = _build_tables(x_p, nt_p, pos_p, w1a, w1b, w1p, be1.reshape(1, HID))
    agg = _edge_kernel(s2d, r2d, ta, tb, We2, cons)
    out = _node_mlp(x_p, nt_p, agg[0], agg[1], cx_p,
                    Wn1[:11], Wn1[11:], bn1.reshape(1, HID), Wo, bo.reshape(1, 2))
    return out[:N]
